# baseline (speedup 1.0000x reference)
import functools

import jax
import jax.numpy as jnp
from jax import lax
from jax.experimental import pallas as pl
from jax.experimental.pallas import tpu as pltpu
from jax.experimental.pallas import tpu_sc as plsc

NC = 2
NS = 16
NW = NC * NS
EB = 128


def _mesh():
    return plsc.VectorSubcoreMesh(
        core_axis_name="c", subcore_axis_name="s", num_cores=NC, num_subcores=NS
    )


def _wid():
    return lax.axis_index("s") * NC + lax.axis_index("c")


def _degree_sc(dstb, nacc):
    nblk = dstb.shape[0]
    half = nblk // NW // 2
    zeros = jnp.zeros((nacc,), jnp.float32)

    @functools.partial(
        pl.kernel,
        out_type=jax.ShapeDtypeStruct((NC, nacc), jnp.float32),
        mesh=_mesh(),
        scratch_types=[
            pltpu.VMEM((1, EB), jnp.int32),
            pltpu.VMEM((1, EB), jnp.int32),
            pltpu.VMEM((EB,), jnp.float32),
            pltpu.VMEM_SHARED((nacc,), jnp.float32),
            pltpu.SemaphoreType.DMA,
            pltpu.SemaphoreType.DMA,
        ],
    )
    def deg_kernel(dstb_hbm, zero_hbm, out_hbm, idx0, idx1, ones_v, acc_sh,
                   ss0, ss1):
        c = lax.axis_index("c")
        sid = lax.axis_index("s")
        wid = _wid()
        for g in range(EB // 16):
            ones_v[pl.ds(g * 16, 16)] = jnp.ones((16,), jnp.float32)

        @pl.when(sid == 0)
        def _():
            pltpu.sync_copy(zero_hbm, acc_sh)

        plsc.subcore_barrier()

        pltpu.sync_copy(dstb_hbm.at[wid], idx0.at[0])
        pltpu.async_copy(ones_v, acc_sh.at[idx0.at[0]], ss0, add=True)

        def body(g, _):
            blk_o = (2 * g + 1) * NW + wid
            blk_n = (2 * g + 2) * NW + wid

            @pl.when(g > 0)
            def _():
                pltpu.make_async_copy(ones_v, acc_sh.at[idx1.at[0]], ss1).wait()

            pltpu.sync_copy(dstb_hbm.at[blk_o], idx1.at[0])
            pltpu.async_copy(ones_v, acc_sh.at[idx1.at[0]], ss1, add=True)
            pltpu.make_async_copy(ones_v, acc_sh.at[idx0.at[0]], ss0).wait()

            @pl.when(g < half - 1)
            def _():
                pltpu.sync_copy(dstb_hbm.at[blk_n], idx0.at[0])
                pltpu.async_copy(ones_v, acc_sh.at[idx0.at[0]], ss0, add=True)

            return ()

        lax.fori_loop(0, half, body, ())
        pltpu.make_async_copy(ones_v, acc_sh.at[idx1.at[0]], ss1).wait()
        plsc.subcore_barrier()

        @pl.when(sid == 0)
        def _():
            pltpu.sync_copy(acc_sh, out_hbm.at[c])

    return deg_kernel(dstb, zeros)


def _segsum_sc(u, sdb, nacc, rows_per_tile):
    nblk = sdb.shape[0]
    H = u.shape[1]
    half = nblk // NW // 2
    zeros = jnp.zeros((rows_per_tile, H), jnp.float32)

    @functools.partial(
        pl.kernel,
        out_type=jax.ShapeDtypeStruct((NC, nacc, H), jnp.float32),
        mesh=_mesh(),
        scratch_types=[
            pltpu.VMEM((2, EB), jnp.int32),
            pltpu.VMEM((2, EB), jnp.int32),
            pltpu.VMEM((EB, H), jnp.float32),
            pltpu.VMEM((EB, H), jnp.float32),
            pltpu.VMEM_SHARED((nacc, H), jnp.float32),
            pltpu.SemaphoreType.DMA,
            pltpu.SemaphoreType.DMA,
            pltpu.SemaphoreType.DMA,
            pltpu.SemaphoreType.DMA,
        ],
    )
    def seg_kernel(u_hbm, sdb_hbm, zero_hbm, out_hbm,
                   sd0, sd1, rows0, rows1, acc_sh, gs0, gs1, ss0, ss1):
        c = lax.axis_index("c")
        sid = lax.axis_index("s")
        wid = _wid()
        base = sid * rows_per_tile
        pltpu.sync_copy(zero_hbm, acc_sh.at[pl.ds(base, rows_per_tile)])
        plsc.subcore_barrier()

        pltpu.sync_copy(sdb_hbm.at[wid], sd0)
        pltpu.async_copy(u_hbm.at[sd0.at[0]], rows0, gs0)

        def body(g, _):
            blk_o = (2 * g + 1) * NW + wid
            blk_n = (2 * g + 2) * NW + wid

            @pl.when(g > 0)
            def _():
                pltpu.make_async_copy(rows1, acc_sh.at[sd1.at[1]], ss1).wait()

            pltpu.sync_copy(sdb_hbm.at[blk_o], sd1)
            pltpu.async_copy(u_hbm.at[sd1.at[0]], rows1, gs1)
            pltpu.make_async_copy(u_hbm.at[sd0.at[0]], rows0, gs0).wait()
            pltpu.async_copy(rows0, acc_sh.at[sd0.at[1]], ss0, add=True)

            pltpu.make_async_copy(rows0, acc_sh.at[sd0.at[1]], ss0).wait()

            @pl.when(g < half - 1)
            def _():
                pltpu.sync_copy(sdb_hbm.at[blk_n], sd0)
                pltpu.async_copy(u_hbm.at[sd0.at[0]], rows0, gs0)

            pltpu.make_async_copy(u_hbm.at[sd1.at[0]], rows1, gs1).wait()
            pltpu.async_copy(rows1, acc_sh.at[sd1.at[1]], ss1, add=True)
            return ()

        lax.fori_loop(0, half, body, ())
        pltpu.make_async_copy(rows1, acc_sh.at[sd1.at[1]], ss1).wait()
        plsc.subcore_barrier()
        pltpu.sync_copy(acc_sh.at[pl.ds(base, rows_per_tile)],
                        out_hbm.at[c, pl.ds(base, rows_per_tile)])

    return seg_kernel(u, sdb, zeros)


def _pairs_sc(a, h2, ijb):
    nblk = ijb.shape[0]
    H = a.shape[1]
    G = H // 16
    half = nblk // NW // 2

    @functools.partial(
        pl.kernel,
        out_type=jax.ShapeDtypeStruct((nblk, EB, 16), jnp.float32),
        mesh=_mesh(),
        scratch_types=[
            pltpu.VMEM((2, EB), jnp.int32),
            pltpu.VMEM((2, EB), jnp.int32),
            pltpu.VMEM((EB, H), jnp.float32),
            pltpu.VMEM((EB, H), jnp.float32),
            pltpu.VMEM((EB, H), jnp.float32),
            pltpu.VMEM((EB, H), jnp.float32),
            pltpu.VMEM((EB, 16), jnp.float32),
            pltpu.VMEM((EB, 16), jnp.float32),
            pltpu.SemaphoreType.DMA,
            pltpu.SemaphoreType.DMA,
            pltpu.SemaphoreType.DMA,
            pltpu.SemaphoreType.DMA,
            pltpu.SemaphoreType.DMA,
            pltpu.SemaphoreType.DMA,
        ],
    )
    def pair_kernel(a_hbm, h2_hbm, ijb_hbm, out_hbm,
                    ij0, ij1, va0, vb0, va1, vb1, part0, part1,
                    ga0, gb0, ga1, gb1, os0, os1):
        wid = _wid()

        def compute(va, vb, part):
            def pbody(p, _):
                acc = va[p, pl.ds(0, 16)] * vb[p, pl.ds(0, 16)]
                for g in range(1, G):
                    acc += va[p, pl.ds(g * 16, 16)] * vb[p, pl.ds(g * 16, 16)]
                part[p, :] = acc
                return ()

            lax.fori_loop(0, EB, pbody, ())

        pltpu.sync_copy(ijb_hbm.at[wid], ij0)
        pltpu.async_copy(a_hbm.at[ij0.at[0]], va0, ga0)
        pltpu.async_copy(h2_hbm.at[ij0.at[1]], vb0, gb0)

        def body(g, _):
            blk_e = (2 * g) * NW + wid
            blk_o = (2 * g + 1) * NW + wid
            blk_n = (2 * g + 2) * NW + wid

            pltpu.sync_copy(ijb_hbm.at[blk_o], ij1)
            pltpu.async_copy(a_hbm.at[ij1.at[0]], va1, ga1)
            pltpu.async_copy(h2_hbm.at[ij1.at[1]], vb1, gb1)
            pltpu.make_async_copy(a_hbm.at[ij0.at[0]], va0, ga0).wait()
            pltpu.make_async_copy(h2_hbm.at[ij0.at[1]], vb0, gb0).wait()

            @pl.when(g > 0)
            def _():
                pltpu.make_async_copy(part0, out_hbm.at[blk_e], os0).wait()

            compute(va0, vb0, part0)
            pltpu.async_copy(part0, out_hbm.at[blk_e], os0)

            @pl.when(g < half - 1)
            def _():
                pltpu.sync_copy(ijb_hbm.at[blk_n], ij0)
                pltpu.async_copy(a_hbm.at[ij0.at[0]], va0, ga0)
                pltpu.async_copy(h2_hbm.at[ij0.at[1]], vb0, gb0)

            pltpu.make_async_copy(a_hbm.at[ij1.at[0]], va1, ga1).wait()
            pltpu.make_async_copy(h2_hbm.at[ij1.at[1]], vb1, gb1).wait()

            @pl.when(g > 0)
            def _():
                pltpu.make_async_copy(part1, out_hbm.at[blk_o], os1).wait()

            compute(va1, vb1, part1)
            pltpu.async_copy(part1, out_hbm.at[blk_o], os1)
            return ()

        lax.fori_loop(0, half, body, ())
        last_e = (2 * (half - 1)) * NW + wid
        last_o = (2 * (half - 1) + 1) * NW + wid
        pltpu.make_async_copy(part0, out_hbm.at[last_e], os0).wait()
        pltpu.make_async_copy(part1, out_hbm.at[last_o], os1).wait()

    return pair_kernel(a, h2, ijb)


def _dense1_tc(x, deg2, WinT, b_in2, W1T):
    N, D = x.shape
    H = WinT.shape[1]
    BR = 1000
    grid = (N // BR,)

    def body(x_ref, deg_ref, winT_ref, b_ref, w1T_ref, out_ref):
        dinv = lax.rsqrt(deg_ref[:, 0] + deg_ref[:, 1] + 1.0)
        h0 = jnp.dot(x_ref[...], winT_ref[...], preferred_element_type=jnp.float32)
        h0 = jnp.maximum(h0 + b_ref[...], 0.0)
        u1 = jnp.dot(h0, w1T_ref[...], preferred_element_type=jnp.float32)
        out_ref[...] = u1 * dinv[:, None]

    return pl.pallas_call(
        body,
        grid=grid,
        in_specs=[
            pl.BlockSpec((BR, D), lambda i: (i, 0)),
            pl.BlockSpec((BR, 2), lambda i: (i, 0)),
            pl.BlockSpec((D, H), lambda i: (0, 0)),
            pl.BlockSpec((1, H), lambda i: (0, 0)),
            pl.BlockSpec((H, H), lambda i: (0, 0)),
        ],
        out_specs=pl.BlockSpec((BR, H), lambda i: (i, 0)),
        out_shape=jax.ShapeDtypeStruct((N, H), jnp.float32),
    )(x, deg2, WinT, b_in2, W1T)


def _dense2_tc(acc, u1, deg2, b12, W2T):
    N, H = u1.shape
    BR = 1000
    grid = (N // BR,)

    def body(acc_ref, u_ref, deg_ref, b_ref, w2T_ref, out_ref):
        dinv = lax.rsqrt(deg_ref[:, 0] + deg_ref[:, 1] + 1.0)
        s = acc_ref[0] + acc_ref[1] + u_ref[...]
        h1 = jnp.maximum(s * dinv[:, None] + b_ref[...], 0.0)
        u2 = jnp.dot(h1, w2T_ref[...], preferred_element_type=jnp.float32)
        out_ref[...] = u2 * dinv[:, None]

    return pl.pallas_call(
        body,
        grid=grid,
        in_specs=[
            pl.BlockSpec((2, BR, H), lambda i: (0, i, 0)),
            pl.BlockSpec((BR, H), lambda i: (i, 0)),
            pl.BlockSpec((BR, 2), lambda i: (i, 0)),
            pl.BlockSpec((1, H), lambda i: (0, 0)),
            pl.BlockSpec((H, H), lambda i: (0, 0)),
        ],
        out_specs=pl.BlockSpec((BR, H), lambda i: (i, 0)),
        out_shape=jax.ShapeDtypeStruct((N, H), jnp.float32),
    )(acc, u1, deg2, b12, W2T)


def _dense3_tc(acc, u2, deg2, b22, dm):
    N, H = u2.shape
    BR = 1000
    grid = (N // BR,)

    def body(acc_ref, u_ref, deg_ref, b_ref, dm_ref, h2_ref, a_ref):
        dinv = lax.rsqrt(deg_ref[:, 0] + deg_ref[:, 1] + 1.0)
        s = acc_ref[0] + acc_ref[1] + u_ref[...]
        h2 = s * dinv[:, None] + b_ref[...]
        h2_ref[...] = h2
        a_ref[...] = h2 * dm_ref[...]

    return pl.pallas_call(
        body,
        grid=grid,
        in_specs=[
            pl.BlockSpec((2, BR, H), lambda i: (0, i, 0)),
            pl.BlockSpec((BR, H), lambda i: (i, 0)),
            pl.BlockSpec((BR, 2), lambda i: (i, 0)),
            pl.BlockSpec((1, H), lambda i: (0, 0)),
            pl.BlockSpec((1, H), lambda i: (0, 0)),
        ],
        out_specs=[
            pl.BlockSpec((BR, H), lambda i: (i, 0)),
            pl.BlockSpec((BR, H), lambda i: (i, 0)),
        ],
        out_shape=[
            jax.ShapeDtypeStruct((N, H), jnp.float32),
            jax.ShapeDtypeStruct((N, H), jnp.float32),
        ],
    )(acc, u2, deg2, b22, dm)


def _reduce_tc(part):
    nblk, eb, L = part.shape
    BR = 16
    grid = (nblk // BR,)

    def body(p_ref, out_ref):
        out_ref[...] = jnp.sum(p_ref[...], axis=-1)

    return pl.pallas_call(
        body,
        grid=grid,
        in_specs=[pl.BlockSpec((BR, eb, L), lambda i: (i, 0, 0))],
        out_specs=pl.BlockSpec((BR, eb), lambda i: (i, 0)),
        out_shape=jax.ShapeDtypeStruct((nblk, eb), jnp.float32),
    )(part)


def kernel(x, edge_index, node_i, node_j, W_in, b_in, distmult, W1, b1, W2, b2):
    N, D = x.shape
    H = W_in.shape[0]
    E = edge_index.shape[1]
    P = node_i.shape[0]

    rows_per_tile = (-(-N // NS) + 7) // 8 * 8
    nacc = rows_per_tile * NS

    nblk_e = -(-(-(-E // EB)) // (2 * NW)) * 2 * NW
    epad = nblk_e * EB - E
    src = jnp.concatenate([edge_index[0], jnp.zeros((epad,), jnp.int32)])
    dst = jnp.concatenate([
        edge_index[1],
        N + (jnp.arange(epad, dtype=jnp.int32) % (nacc - N)),
    ])
    srcb = src.reshape(-1, EB)
    dstb = dst.reshape(-1, EB)
    sdb = jnp.stack([srcb, dstb], axis=1)

    deg2 = _degree_sc(dstb, nacc)[:, :N].T

    WinT = W_in.T
    W1T = W1.T
    W2T = W2.T
    b_in2 = b_in.reshape(1, H)
    b12 = b1.reshape(1, H)
    b22 = b2.reshape(1, H)

    zpad = jnp.zeros((nacc - N, H), jnp.float32)
    u1 = _dense1_tc(x, deg2, WinT, b_in2, W1T)
    acc1 = _segsum_sc(jnp.concatenate([u1, zpad]), sdb, nacc,
                      rows_per_tile)[:, :N, :]

    u2 = _dense2_tc(acc1, u1, deg2, b12, W2T)
    acc2 = _segsum_sc(jnp.concatenate([u2, zpad]), sdb, nacc,
                      rows_per_tile)[:, :N, :]

    h2, a = _dense3_tc(acc2, u2, deg2, b22, distmult)

    nblk_p = -(-(-(-P // EB)) // (2 * NW)) * 2 * NW
    ppad = nblk_p * EB - P
    ib = jnp.concatenate([node_i, jnp.zeros((ppad,), jnp.int32)]).reshape(-1, EB)
    jb = jnp.concatenate([node_j, jnp.zeros((ppad,), jnp.int32)]).reshape(-1, EB)
    ijb = jnp.stack([ib, jb], axis=1)

    part = _pairs_sc(a, h2, ijb)
    scores = _reduce_tc(part).reshape(-1)[:P]
    return scores
